# baseline (speedup 1.0000x reference)
import functools

import jax
import jax.numpy as jnp
from jax import lax
from jax.experimental import pallas as pl
from jax.experimental.pallas import tpu as pltpu
from jax.experimental.pallas import tpu_sc as plsc

BATCH = 4
NUM_POSITIONS = 8192
FEATURE_DIM = 768

_NC = 2
_NS = 16
_NW = _NC * _NS
_SPAN = NUM_POSITIONS // _NW
_C = 16
_CHUNKS = _SPAN // _C
_XROWS = BATCH * NUM_POSITIONS


def _sc_body(x_hbm, e_hbm, o_hbm, xbufs, obufs, ebufs, lsems, ssems, esems):
    wid = lax.axis_index("s") * _NC + lax.axis_index("c")
    pos_base = wid * _SPAN

    def xo_row(chunk, b):
        row = b * NUM_POSITIONS + pos_base + chunk * _C
        return jnp.minimum(row, _XROWS - _C)

    def e_load(chunk, cc):
        row = jnp.minimum(pos_base + chunk * _C, NUM_POSITIONS - _C)
        return pltpu.async_copy(e_hbm.at[pl.ds(row, _C)], ebufs[cc], esems[cc])

    def x_load(chunk, b, s):
        return pltpu.async_copy(
            x_hbm.at[pl.ds(xo_row(chunk, b), _C)], xbufs[s], lsems[s])

    def o_store(chunk, b, s):
        return pltpu.async_copy(
            obufs[s], o_hbm.at[pl.ds(xo_row(chunk, b), _C)], ssems[s])

    def wait_x(s):
        pltpu.make_async_copy(x_hbm.at[pl.ds(0, _C)], xbufs[s], lsems[s]).wait()

    def wait_e(cc):
        pltpu.make_async_copy(e_hbm.at[pl.ds(0, _C)], ebufs[cc], esems[cc]).wait()

    def wait_s(s):
        pltpu.make_async_copy(obufs[s], o_hbm.at[pl.ds(0, _C)], ssems[s]).wait()

    e_load(0, 0)
    e_load(1, 1)
    x_load(0, 0, 0)
    x_load(0, 1, 1)
    o_store(0, 0, 0)
    o_store(0, 1, 1)

    def pair_body(k, carry):
        for cc in range(2):
            c = 2 * k + cc
            wait_e(cc)
            for u in range(BATCH):
                t_par2 = u % 2
                nc, nb = (c, u + 2) if u < 2 else (c + 1, u - 2)
                x_load(nc, nb, (u + 2) % 4)
                wait_x(u)
                wait_s(t_par2)
                xbuf, obuf, ebuf = xbufs[u], obufs[t_par2], ebufs[cc]

                @plsc.parallel_loop(0, FEATURE_DIM, 16, unroll=2)
                def _add(i):
                    for r in range(_C):
                        obuf[r, pl.ds(i, 16)] = (
                            xbuf[r, pl.ds(i, 16)] + ebuf[r, pl.ds(i, 16)])

                o_store(c, u, t_par2)
            e_load(c + 2, cc)
        return carry

    lax.fori_loop(0, _CHUNKS // 2, pair_body, 0)

    wait_x(0)
    wait_x(1)
    wait_e(0)
    wait_e(1)
    wait_s(0)
    wait_s(1)


@functools.partial(
    pl.kernel,
    out_type=jax.ShapeDtypeStruct((BATCH * NUM_POSITIONS, FEATURE_DIM), jnp.float32),
    mesh=plsc.VectorSubcoreMesh(core_axis_name="c", subcore_axis_name="s"),
    scratch_types=[
        [pltpu.VMEM((_C, FEATURE_DIM), jnp.float32) for _ in range(BATCH)],
        [pltpu.VMEM((_C, FEATURE_DIM), jnp.float32) for _ in range(2)],
        [pltpu.VMEM((_C, FEATURE_DIM), jnp.float32) for _ in range(2)],
        [pltpu.SemaphoreType.DMA for _ in range(BATCH)],
        [pltpu.SemaphoreType.DMA for _ in range(2)],
        [pltpu.SemaphoreType.DMA for _ in range(2)],
    ],
)
def _sc_kernel(x_hbm, e_hbm, o_hbm, xbufs, obufs, ebufs, lsems, ssems, esems):
    _sc_body(x_hbm, e_hbm, o_hbm, xbufs, obufs, ebufs, lsems, ssems, esems)


def kernel(x, embedding):
    x2d = x.reshape(BATCH * NUM_POSITIONS, FEATURE_DIM)
    out = _sc_kernel(x2d, embedding)
    return out.reshape(BATCH, NUM_POSITIONS, FEATURE_DIM)


# speedup vs baseline: 1.0603x; 1.0603x over previous
import functools

import jax
import jax.numpy as jnp
from jax import lax
from jax.experimental import pallas as pl
from jax.experimental.pallas import tpu as pltpu
from jax.experimental.pallas import tpu_sc as plsc

BATCH = 4
NUM_POSITIONS = 8192
FEATURE_DIM = 768

_NC = 2
_NS = 16
_NW = _NC * _NS
_SPAN = NUM_POSITIONS // _NW
_C = 16
_CHUNKS = _SPAN // _C
_XROWS = BATCH * NUM_POSITIONS


def _sc_body(x_hbm, e_hbm, o_hbm, xbufs, obufs, ebufs, lsems, ssems, esems):
    wid = lax.axis_index("s") * _NC + lax.axis_index("c")
    pos_base = wid * _SPAN

    def xo_row(chunk, b):
        row = b * NUM_POSITIONS + pos_base + chunk * _C
        return jnp.minimum(row, _XROWS - _C)

    def e_load(chunk, cc):
        row = jnp.minimum(pos_base + chunk * _C, NUM_POSITIONS - _C)
        return pltpu.async_copy(e_hbm.at[pl.ds(row, _C)], ebufs[cc], esems[cc])

    def x_load(chunk, b, s):
        return pltpu.async_copy(
            x_hbm.at[pl.ds(xo_row(chunk, b), _C)], xbufs[s], lsems[s])

    def o_store(chunk, b, s):
        return pltpu.async_copy(
            obufs[s], o_hbm.at[pl.ds(xo_row(chunk, b), _C)], ssems[s])

    def wait_x(s):
        pltpu.make_async_copy(x_hbm.at[pl.ds(0, _C)], xbufs[s], lsems[s]).wait()

    def wait_e(cc):
        pltpu.make_async_copy(e_hbm.at[pl.ds(0, _C)], ebufs[cc], esems[cc]).wait()

    def wait_s(s):
        pltpu.make_async_copy(obufs[s], o_hbm.at[pl.ds(0, _C)], ssems[s]).wait()

    e_load(0, 0)
    e_load(1, 1)
    x_load(0, 0, 0)
    x_load(0, 1, 1)
    o_store(0, 0, 0)
    o_store(0, 1, 1)

    def pair_body(k, carry):
        for cc in range(2):
            c = 2 * k + cc
            wait_e(cc)
            for u in range(BATCH):
                t_par2 = u % 2
                nc, nb = (c, u + 2) if u < 2 else (c + 1, u - 2)
                x_load(nc, nb, (u + 2) % 4)
                wait_x(u)
                wait_s(t_par2)
                xbuf, obuf, ebuf = xbufs[u], obufs[t_par2], ebufs[cc]

                @plsc.parallel_loop(0, FEATURE_DIM, 16)
                def _add(i):
                    for r in range(_C):
                        obuf[r, pl.ds(i, 16)] = (
                            xbuf[r, pl.ds(i, 16)] + ebuf[r, pl.ds(i, 16)])

                o_store(c, u, t_par2)
            e_load(c + 2, cc)
        return carry

    lax.fori_loop(0, _CHUNKS // 2, pair_body, 0)

    wait_x(0)
    wait_x(1)
    wait_e(0)
    wait_e(1)
    wait_s(0)
    wait_s(1)


@functools.partial(
    pl.kernel,
    out_type=jax.ShapeDtypeStruct((BATCH * NUM_POSITIONS, FEATURE_DIM), jnp.float32),
    mesh=plsc.VectorSubcoreMesh(core_axis_name="c", subcore_axis_name="s"),
    scratch_types=[
        [pltpu.VMEM((_C, FEATURE_DIM), jnp.float32) for _ in range(BATCH)],
        [pltpu.VMEM((_C, FEATURE_DIM), jnp.float32) for _ in range(2)],
        [pltpu.VMEM((_C, FEATURE_DIM), jnp.float32) for _ in range(2)],
        [pltpu.SemaphoreType.DMA for _ in range(BATCH)],
        [pltpu.SemaphoreType.DMA for _ in range(2)],
        [pltpu.SemaphoreType.DMA for _ in range(2)],
    ],
)
def _sc_kernel(x_hbm, e_hbm, o_hbm, xbufs, obufs, ebufs, lsems, ssems, esems):
    _sc_body(x_hbm, e_hbm, o_hbm, xbufs, obufs, ebufs, lsems, ssems, esems)


def kernel(x, embedding):
    x2d = x.reshape(BATCH * NUM_POSITIONS, FEATURE_DIM)
    out = _sc_kernel(x2d, embedding)
    return out.reshape(BATCH, NUM_POSITIONS, FEATURE_DIM)
